# baseline (speedup 1.0000x reference)
import functools

import jax
import jax.numpy as jnp
from jax import lax
from jax.experimental import pallas as pl
from jax.experimental.pallas import tpu as pltpu, tpu_sc as plsc

B = 16384
D = 128
G = 8
N_GENRES = 100
T_ROWS = 1001

NC = 2
NS = 16
NW = NC * NS
BPW = B // NW
L = 16
IDX_CHUNKS = BPW // 128


def _sc_body(t_idx_hbm, genres_hbm, t_table_hbm, style_hbm, out_hbm,
             idx_v, gv_v, style_v, rows_v, sem):
    wid = lax.axis_index("s") * NC + lax.axis_index("c")
    base = wid * BPW

    pltpu.sync_copy(t_idx_hbm.at[pl.ds(wid * IDX_CHUNKS, IDX_CHUNKS), :], idx_v)
    pltpu.sync_copy(genres_hbm.at[pl.ds(base * G, BPW * G)], gv_v)
    pltpu.sync_copy(style_hbm, style_v)

    for j in range(IDX_CHUNKS):
        pltpu.async_copy(t_table_hbm.at[idx_v.at[j]],
                         rows_v.at[pl.ds(j * 128, 128), :], sem).wait()
    pltpu.sync_copy(rows_v, out_hbm.at[pl.ds(base, BPW), pl.ds(0, D)])

    cols = lax.iota(jnp.int32, L)
    col_chunks = [cols + c * L for c in range(D // L)]

    @plsc.parallel_loop(0, BPW, unroll=4)
    def _style_loop(i):
        gvals = [plsc.load_gather(gv_v, [jnp.full((L,), i * G + g, jnp.int32)])
                 for g in range(G)]
        row_ref = rows_v.at[i]
        for c in range(D // L):
            acc = plsc.load_gather(style_v, [gvals[0], col_chunks[c]])
            for g in range(1, G):
                acc = acc + plsc.load_gather(style_v, [gvals[g], col_chunks[c]])
            row_ref[pl.ds(c * L, L)] = acc
    pltpu.sync_copy(rows_v, out_hbm.at[pl.ds(base, BPW), pl.ds(D, D)])


_sc_call = functools.partial(
    pl.kernel,
    out_type=jax.ShapeDtypeStruct((B, 2 * D), jnp.float32),
    mesh=plsc.VectorSubcoreMesh(core_axis_name="c", subcore_axis_name="s"),
    compiler_params=pltpu.CompilerParams(needs_layout_passes=False),
    scratch_types=[
        pltpu.VMEM((IDX_CHUNKS, 128), jnp.int32),
        pltpu.VMEM((BPW * G,), jnp.int32),
        pltpu.VMEM((N_GENRES, D), jnp.float32),
        pltpu.VMEM((BPW, D), jnp.float32),
        pltpu.SemaphoreType.DMA,
    ],
)(_sc_body)


def kernel(t, length, genres, t_table, style_table):
    del length
    t_idx = t.astype(jnp.int32).reshape(B // 128, 128)
    genres_flat = genres.astype(jnp.int32).reshape(-1)
    out = _sc_call(t_idx, genres_flat, t_table, style_table)
    return out.reshape(B, 2 * D, 1)


# speedup vs baseline: 1.4375x; 1.4375x over previous
import functools

import jax
import jax.numpy as jnp
from jax import lax
from jax.experimental import pallas as pl
from jax.experimental.pallas import tpu as pltpu, tpu_sc as plsc

B = 16384
D = 128
G = 8
N_GENRES = 100
T_ROWS = 1001

NC = 2
NS = 16
NW = NC * NS
BPW = B // NW
L = 16
IDX_CHUNKS = BPW // 128


def _sc_body(t_idx_hbm, genres_hbm, t_table_hbm, style_hbm, out_hbm,
             idx_v, gv_v, style_v, rows_v, sem):
    wid = lax.axis_index("s") * NC + lax.axis_index("c")
    base = wid * BPW

    pltpu.sync_copy(t_idx_hbm.at[pl.ds(wid * IDX_CHUNKS, IDX_CHUNKS), :], idx_v)
    pltpu.sync_copy(genres_hbm.at[pl.ds(base * G, BPW * G)], gv_v)
    pltpu.sync_copy(style_hbm, style_v)

    for j in range(IDX_CHUNKS):
        pltpu.async_copy(t_table_hbm.at[idx_v.at[j]],
                         rows_v.at[pl.ds(j * 128, 128), :], sem).wait()
    pltpu.sync_copy(rows_v, out_hbm.at[pl.ds(base, BPW), pl.ds(0, D)])

    cols = lax.iota(jnp.int32, L)
    col_chunks = [cols + c * L for c in range(D // L)]

    @plsc.parallel_loop(0, BPW, unroll=2)
    def _style_loop(i):
        gvals = [plsc.load_gather(gv_v, [jnp.full((L,), i * G + g, jnp.int32)])
                 for g in range(G)]
        row_ref = rows_v.at[i]
        for c in range(D // L):
            acc = plsc.load_gather(style_v, [gvals[0], col_chunks[c]])
            for g in range(1, G):
                acc = acc + plsc.load_gather(style_v, [gvals[g], col_chunks[c]])
            row_ref[pl.ds(c * L, L)] = acc
    pltpu.sync_copy(rows_v, out_hbm.at[pl.ds(base, BPW), pl.ds(D, D)])


_sc_call = functools.partial(
    pl.kernel,
    out_type=jax.ShapeDtypeStruct((B, 2 * D), jnp.float32),
    mesh=plsc.VectorSubcoreMesh(core_axis_name="c", subcore_axis_name="s"),
    compiler_params=pltpu.CompilerParams(needs_layout_passes=False),
    scratch_types=[
        pltpu.VMEM((IDX_CHUNKS, 128), jnp.int32),
        pltpu.VMEM((BPW * G,), jnp.int32),
        pltpu.VMEM((N_GENRES, D), jnp.float32),
        pltpu.VMEM((BPW, D), jnp.float32),
        pltpu.SemaphoreType.DMA,
    ],
)(_sc_body)


def kernel(t, length, genres, t_table, style_table):
    del length
    t_idx = t.astype(jnp.int32).reshape(B // 128, 128)
    genres_flat = genres.astype(jnp.int32).reshape(-1)
    out = _sc_call(t_idx, genres_flat, t_table, style_table)
    return out.reshape(B, 2 * D, 1)


# speedup vs baseline: 1.4992x; 1.0429x over previous
import functools

import jax
import jax.numpy as jnp
from jax import lax
from jax.experimental import pallas as pl
from jax.experimental.pallas import tpu as pltpu, tpu_sc as plsc

B = 16384
D = 128
G = 8
N_GENRES = 100
T_ROWS = 1001

NC = 2
NS = 16
NW = NC * NS
BPW = B // NW
L = 16
IDX_CHUNKS = BPW // 128


def _sc_body(t_idx_hbm, genres_hbm, t_table_hbm, style_hbm, out_hbm,
             idx_v, gv_v, style_v, rows_v, sem):
    wid = lax.axis_index("s") * NC + lax.axis_index("c")
    base = wid * BPW

    pltpu.sync_copy(t_idx_hbm.at[pl.ds(wid * IDX_CHUNKS, IDX_CHUNKS), :], idx_v)
    pltpu.sync_copy(genres_hbm.at[pl.ds(base * G, BPW * G)], gv_v)
    pltpu.sync_copy(style_hbm, style_v)

    for j in range(IDX_CHUNKS):
        pltpu.async_copy(t_table_hbm.at[idx_v.at[j]],
                         rows_v.at[pl.ds(j * 128, 128), :], sem).wait()
    pltpu.sync_copy(rows_v, out_hbm.at[pl.ds(base, BPW), pl.ds(0, D)])

    cols = lax.iota(jnp.int32, L)
    col_chunks = [cols + c * L for c in range(D // L)]

    @plsc.parallel_loop(0, BPW, unroll=2)
    def _style_loop(i):
        gvals = [plsc.load_gather(gv_v, [jnp.full((L,), i * G + g, jnp.int32)])
                 for g in range(G)]
        row_ref = rows_v.at[i]
        for c in range(D // L):
            acc = plsc.load_gather(style_v, [gvals[0], col_chunks[c]])
            for g in range(1, G):
                acc = acc + plsc.load_gather(style_v, [gvals[g], col_chunks[c]])
            row_ref[pl.ds(c * L, L)] = acc
    pltpu.sync_copy(rows_v, out_hbm.at[pl.ds(base, BPW), pl.ds(D, D)])


_sc_call = functools.partial(
    pl.kernel,
    out_type=jax.ShapeDtypeStruct((B, 2 * D), jnp.float32),
    mesh=plsc.VectorSubcoreMesh(core_axis_name="c", subcore_axis_name="s"),
    compiler_params=pltpu.CompilerParams(
        needs_layout_passes=False, use_tc_tiling_on_sc=False),
    scratch_types=[
        pltpu.VMEM((IDX_CHUNKS, 128), jnp.int32),
        pltpu.VMEM((BPW * G,), jnp.int32),
        pltpu.VMEM((N_GENRES, D), jnp.float32),
        pltpu.VMEM((BPW, D), jnp.float32),
        pltpu.SemaphoreType.DMA,
    ],
)(_sc_body)


def kernel(t, length, genres, t_table, style_table):
    del length
    t_idx = t.astype(jnp.int32).reshape(B // 128, 128)
    genres_flat = genres.astype(jnp.int32).reshape(-1)
    out = _sc_call(t_idx, genres_flat, t_table, style_table)
    return out.reshape(B, 2 * D, 1)
